# baseline (speedup 1.0000x reference)
import functools
import math

import jax
import jax.numpy as jnp
from jax import lax
from jax.experimental import pallas as pl
from jax.experimental.pallas import tpu as pltpu
from jax.experimental.pallas import tpu_sc as plsc

B, S, D = 2, 2048, 1024
H, KV, DH = 16, 8, 64
E, TOPK, F = 8, 2, 2048
THETA, EPS = 10000.0, 1e-6
T = B * S
A = T * TOPK
BLK = 256
NB = (A // BLK) + E - 1
PAD = NB * BLK
RB = 512
NRB = T // RB

HALF = DH // 2
_LOG_THETA = math.log(THETA)


def _qkv_body(x_ref, w_ref, ln_ref, cos_ref, sin_ref, out_ref):
    x = x_ref[...]
    ms = jnp.mean(x * x, axis=-1, keepdims=True)
    xn = x * jax.lax.rsqrt(ms + EPS) * ln_ref[...]
    qkv = jnp.dot(xn.astype(jnp.bfloat16), w_ref[...].astype(jnp.bfloat16),
                  preferred_element_type=jnp.float32)

    QK = (H + KV) * DH
    NH = QK // DH
    qk = qkv[:, :QK]
    cosv = jnp.concatenate([cos_ref[...]] * NH, axis=1)
    sinv = jnp.concatenate([sin_ref[...]] * NH, axis=1)
    col = jax.lax.broadcasted_iota(jnp.int32, (RB, QK), 1)
    rolled_p = jnp.concatenate([qk[:, HALF:], qk[:, :HALF]], axis=1)
    rolled_m = jnp.concatenate([qk[:, -HALF:], qk[:, :-HALF]], axis=1)
    first_half = (col % DH) < HALF
    rh = jnp.where(first_half, -rolled_p, rolled_m)
    qk_rot = qk * cosv + rh * sinv
    out_ref[...] = jnp.concatenate([qk_rot, qkv[:, QK:]], axis=1)


def _qkv_call(x2d, wqkv, ln1_w, cos_t, sin_t):
    nsb = S // RB
    return pl.pallas_call(
        _qkv_body,
        grid=(NRB,),
        in_specs=[
            pl.BlockSpec((RB, D), lambda i: (i, 0)),
            pl.BlockSpec((D, (H + 2 * KV) * DH), lambda i: (0, 0)),
            pl.BlockSpec((1, D), lambda i: (0, 0)),
            pl.BlockSpec((RB, DH), lambda i: (i % nsb, 0)),
            pl.BlockSpec((RB, DH), lambda i: (i % nsb, 0)),
        ],
        out_specs=pl.BlockSpec((RB, (H + 2 * KV) * DH), lambda i: (i, 0)),
        out_shape=jax.ShapeDtypeStruct((T, (H + 2 * KV) * DH), jnp.float32),
    )(x2d, wqkv, ln1_w, cos_t, sin_t)


QBLK = 512
NQB = S // QBLK
KBLK = 512


def _attn_body(q_ref, k_ref, v_ref, o_ref):
    iq = pl.program_id(2)
    q = q_ref[0].astype(jnp.bfloat16)
    lane = jax.lax.broadcasted_iota(jnp.int32, (KBLK, 2 * DH), 1)
    is_v = lane < DH
    is_one = lane == DH

    def vaug(j):
        v = v_ref[0, pl.ds(j * KBLK, KBLK), :]
        vb = jnp.concatenate([v, v], axis=1)
        return jnp.where(is_v, vb, jnp.where(is_one, 1.0, 0.0)
                         ).astype(jnp.bfloat16)

    def scores(j):
        k = k_ref[0, pl.ds(j * KBLK, KBLK), :].astype(jnp.bfloat16)
        s = jax.lax.dot_general(q, k, (((1,), (1,)), ((), ())),
                                preferred_element_type=jnp.float32)
        return s * (1.0 / 8.0)

    def update(s, j, carry):
        m, acc = carry
        mj = jnp.max(s, axis=-1, keepdims=True)
        mn = jnp.maximum(m, mj)
        p = jnp.exp(s - mn)
        pv = jnp.dot(p.astype(jnp.bfloat16), vaug(j),
                     preferred_element_type=jnp.float32)
        return mn, acc * jnp.exp(m - mn) + pv

    def step(j, carry):
        return update(scores(j), j, carry)

    m0 = jnp.full((QBLK, 1), -1e30, jnp.float32)
    a0 = jnp.zeros((QBLK, 2 * DH), jnp.float32)
    nkb = iq * (QBLK // KBLK)
    m, acc = jax.lax.fori_loop(0, nkb, step, (m0, a0))

    sd = scores(nkb)
    qpos = iq * QBLK + jax.lax.broadcasted_iota(jnp.int32, (QBLK, KBLK), 0)
    kloc = nkb * KBLK + jax.lax.broadcasted_iota(jnp.int32, (QBLK, KBLK), 1)
    sd = jnp.where(qpos >= kloc, sd, jnp.float32(-1e9))
    m, acc = update(sd, nkb, (m, acc))

    o_ref[0] = acc[:, :DH] / acc[:, DH:DH + 1]


def _attn_call(q3, k3, v3):
    return pl.pallas_call(
        _attn_body,
        grid=(B, H, NQB),
        in_specs=[
            pl.BlockSpec((1, QBLK, DH), lambda b, h, iq: (h, b * NQB + iq, 0)),
            pl.BlockSpec((1, S, DH), lambda b, h, iq: (h // 2, b, 0)),
            pl.BlockSpec((1, S, DH), lambda b, h, iq: (h // 2, b, 0)),
        ],
        out_specs=pl.BlockSpec((1, QBLK, DH),
                               lambda b, h, iq: (h, b * NQB + iq, 0)),
        out_shape=jax.ShapeDtypeStruct((H, T, DH), jnp.float32),
    )(q3, k3, v3)


def _oproj_body(o_ref, wo_ref, res_ref, ln_ref, wg_ref,
                h1_ref, xn2_ref, logits_ref, route_ref):
    h1 = res_ref[...] + jnp.dot(o_ref[...].astype(jnp.bfloat16),
                                wo_ref[...].astype(jnp.bfloat16),
                                preferred_element_type=jnp.float32)
    h1_ref[...] = h1
    ms = jnp.mean(h1 * h1, axis=-1, keepdims=True)
    xn2 = h1 * jax.lax.rsqrt(ms + EPS) * ln_ref[...]
    xn2_ref[...] = xn2
    logits = jnp.dot(xn2.astype(jnp.bfloat16),
                     wg_ref[...].astype(jnp.bfloat16),
                     preferred_element_type=jnp.float32)
    logits_ref[...] = logits
    lm = jnp.max(logits, axis=-1, keepdims=True)
    ex = jnp.exp(logits - lm)
    probs = ex / jnp.sum(ex, axis=-1, keepdims=True)
    ioe = jax.lax.broadcasted_iota(jnp.int32, (RB, E), 1)
    m1 = jnp.max(probs, axis=-1, keepdims=True)
    e1 = jnp.min(jnp.where(probs == m1, ioe, E), axis=-1, keepdims=True)
    probs2 = jnp.where(ioe == e1, jnp.float32(-1.0), probs)
    m2 = jnp.max(probs2, axis=-1, keepdims=True)
    e2 = jnp.min(jnp.where(probs2 == m2, ioe, E), axis=-1, keepdims=True)
    denom = m1 + m2
    w1 = m1 / denom
    w2 = m2 / denom
    route_ref[...] = jnp.concatenate(
        [e1.astype(jnp.float32), e2.astype(jnp.float32), w1, w2,
         jnp.zeros((RB, 4), jnp.float32)], axis=1)


def _oproj_call(o, wo, res, ln2_w, wg):
    return pl.pallas_call(
        _oproj_body,
        grid=(NRB,),
        in_specs=[
            pl.BlockSpec((RB, H * DH), lambda i: (i, 0)),
            pl.BlockSpec((H * DH, D), lambda i: (0, 0)),
            pl.BlockSpec((RB, D), lambda i: (i, 0)),
            pl.BlockSpec((1, D), lambda i: (0, 0)),
            pl.BlockSpec((D, E), lambda i: (0, 0)),
        ],
        out_specs=[
            pl.BlockSpec((RB, D), lambda i: (i, 0)),
            pl.BlockSpec((RB, D), lambda i: (i, 0)),
            pl.BlockSpec((RB, E), lambda i: (i, 0)),
            pl.BlockSpec((RB, E), lambda i: (i, 0)),
        ],
        out_shape=[
            jax.ShapeDtypeStruct((T, D), jnp.float32),
            jax.ShapeDtypeStruct((T, D), jnp.float32),
            jax.ShapeDtypeStruct((T, E), jnp.float32),
            jax.ShapeDtypeStruct((T, E), jnp.float32),
        ],
    )(o, wo, res, ln2_w, wg)


def _gmm_body(be_ref, xs_ref, w1_ref, w3_ref, w2_ref, ws_ref, y_ref):
    del be_ref
    xs = xs_ref[...].astype(jnp.bfloat16)
    a = jnp.dot(xs, w1_ref[0].astype(jnp.bfloat16),
                preferred_element_type=jnp.float32)
    b = jnp.dot(xs, w3_ref[0].astype(jnp.bfloat16),
                preferred_element_type=jnp.float32)
    h = (a / (1.0 + jnp.exp(-a))) * b
    y = jnp.dot(h.astype(jnp.bfloat16), w2_ref[0].astype(jnp.bfloat16),
                preferred_element_type=jnp.float32)
    y_ref[...] = y * ws_ref[...]


def _gmm_call(xs, w1, w3, w2, ws, block_expert):
    grid_spec = pltpu.PrefetchScalarGridSpec(
        num_scalar_prefetch=1,
        grid=(NB,),
        in_specs=[
            pl.BlockSpec((BLK, D), lambda i, be: (i, 0)),
            pl.BlockSpec((1, D, F), lambda i, be: (be[i], 0, 0)),
            pl.BlockSpec((1, D, F), lambda i, be: (be[i], 0, 0)),
            pl.BlockSpec((1, F, D), lambda i, be: (be[i], 0, 0)),
            pl.BlockSpec((BLK, 1), lambda i, be: (i, 0)),
        ],
        out_specs=pl.BlockSpec((BLK, D), lambda i, be: (i, 0)),
    )
    return pl.pallas_call(
        _gmm_body,
        grid_spec=grid_spec,
        out_shape=jax.ShapeDtypeStruct((PAD, D), jnp.float32),
    )(block_expert, xs, w1, w3, w2, ws)


_SC_NC, _SC_NS = 2, 16
_SC_NW = _SC_NC * _SC_NS
GCH = (PAD // _SC_NW) // 3
TCH = (T // _SC_NW) // 4


def _sc_gather_rows(x_hbm, idx_hbm, out_hbm, idx_v, rows_v, sem):
    wid = lax.axis_index("s") * _SC_NC + lax.axis_index("c")
    base = wid * (PAD // _SC_NW)
    for c in range(3):
        off = base + c * GCH
        pltpu.sync_copy(idx_hbm.at[pl.ds(off, GCH)], idx_v)
        pltpu.async_copy(x_hbm.at[idx_v], rows_v, sem).wait()
        pltpu.sync_copy(rows_v, out_hbm.at[pl.ds(off, GCH)])


def _sc_gather_call(xn2, tok_src):
    mesh = plsc.VectorSubcoreMesh(core_axis_name="c", subcore_axis_name="s")
    f = functools.partial(
        pl.kernel, mesh=mesh,
        out_type=jax.ShapeDtypeStruct((PAD, D), jnp.float32),
        scratch_types=[
            pltpu.VMEM((GCH,), jnp.int32),
            pltpu.VMEM((GCH, D), jnp.float32),
            pltpu.SemaphoreType.DMA,
        ],
    )(_sc_gather_rows)
    return f(xn2, tok_src)


def _sc_combine(h1_hbm, y_hbm, p1_hbm, p2_hbm, out_hbm,
                i1_v, i2_v, r1_v, r2_v, h_v, sem):
    wid = lax.axis_index("s") * _SC_NC + lax.axis_index("c")
    base = wid * (T // _SC_NW)
    for c in range(4):
        off = base + c * TCH
        pltpu.sync_copy(p1_hbm.at[pl.ds(off, TCH)], i1_v)
        pltpu.sync_copy(p2_hbm.at[pl.ds(off, TCH)], i2_v)
        pltpu.async_copy(y_hbm.at[i1_v], r1_v, sem).wait()
        pltpu.async_copy(y_hbm.at[i2_v], r2_v, sem).wait()
        pltpu.sync_copy(h1_hbm.at[pl.ds(off, TCH)], h_v)

        def add_step(i, _):
            r = i // (D // 16)
            col = (i % (D // 16)) * 16
            h_v[r, pl.ds(col, 16)] = (h_v[r, pl.ds(col, 16)]
                                      + r1_v[r, pl.ds(col, 16)]
                                      + r2_v[r, pl.ds(col, 16)])
            return 0

        lax.fori_loop(0, TCH * (D // 16), add_step, 0, unroll=4)
        pltpu.sync_copy(h_v, out_hbm.at[pl.ds(off, TCH)])


def _sc_combine_call(h1, y, p1, p2):
    mesh = plsc.VectorSubcoreMesh(core_axis_name="c", subcore_axis_name="s")
    f = functools.partial(
        pl.kernel, mesh=mesh,
        out_type=jax.ShapeDtypeStruct((T, D), jnp.float32),
        scratch_types=[
            pltpu.VMEM((TCH,), jnp.int32),
            pltpu.VMEM((TCH,), jnp.int32),
            pltpu.VMEM((TCH, D), jnp.float32),
            pltpu.VMEM((TCH, D), jnp.float32),
            pltpu.VMEM((TCH, D), jnp.float32),
            pltpu.SemaphoreType.DMA,
        ],
    )(_sc_combine)
    return f(h1, y, p1, p2)


def _route_metadata(route):
    sel = jnp.stack([route[:, 0], route[:, 1]], axis=1).astype(jnp.int32)
    wts = jnp.stack([route[:, 2], route[:, 3]], axis=1)
    self = sel.reshape(A)
    wflat = wts.reshape(A)
    onehot = (self[:, None] == jnp.arange(E, dtype=jnp.int32)[None, :])
    onehot = onehot.astype(jnp.int32)
    ranks = jnp.cumsum(onehot, axis=0) - onehot
    rank_j = jnp.sum(ranks * onehot, axis=1)
    counts = jnp.sum(onehot, axis=0)
    pcounts = ((counts + BLK - 1) // BLK) * BLK
    poff = jnp.concatenate([jnp.zeros((1,), jnp.int32),
                            jnp.cumsum(pcounts)[:-1].astype(jnp.int32)])
    pos = poff[self] + rank_j
    tok = jnp.arange(A, dtype=jnp.int32) // TOPK
    tok_src = jnp.zeros((PAD,), jnp.int32).at[pos].set(tok)
    ws = jnp.zeros((PAD,), jnp.float32).at[pos].set(wflat)
    block_expert = jnp.repeat(jnp.arange(E, dtype=jnp.int32),
                              pcounts // BLK, total_repeat_length=NB)
    return pos, tok_src, ws, block_expert


def kernel(hidden_states, position_ids, gate_logits, ln1_w, ln2_w,
           Wq, Wk, Wv, Wo, Wg, w1, w3, w2):
    x2d = hidden_states.reshape(T, D)
    wqkv = jnp.concatenate([Wq, Wk, Wv], axis=1)

    inv = 1.0 / (THETA ** (jnp.arange(0, DH, 2, dtype=jnp.float32) / DH))
    ang = jnp.arange(S, dtype=jnp.float32)[:, None] * inv[None, :]
    cos_t = jnp.concatenate([jnp.cos(ang), jnp.cos(ang)], axis=1)
    sin_t = jnp.concatenate([jnp.sin(ang), jnp.sin(ang)], axis=1)

    qkv = _qkv_call(x2d, wqkv, ln1_w.reshape(1, D), cos_t, sin_t)
    q3 = qkv[:, :H * DH].reshape(T, H, DH).transpose(1, 0, 2)
    k3 = qkv[:, H * DH:(H + KV) * DH].reshape(T, KV, DH).transpose(1, 0, 2)
    v3 = qkv[:, (H + KV) * DH:].reshape(T, KV, DH).transpose(1, 0, 2)
    o3 = _attn_call(q3, k3, v3)
    o = o3.transpose(1, 0, 2).reshape(T, H * DH)
    h1, xn2, logits, route = _oproj_call(o, Wo, x2d, ln2_w.reshape(1, D), Wg)

    pos, tok_src, ws, block_expert = _route_metadata(route)
    xs = _sc_gather_call(xn2, tok_src)
    y = _gmm_call(xs, w1, w3, w2, ws.reshape(PAD, 1), block_expert)

    p = pos.reshape(T, TOPK)
    out2d = _sc_combine_call(h1, y, p[:, 0], p[:, 1])

    out = out2d.reshape(B, S, D)
    new_gate = gate_logits.at[0].set(logits)
    return (out, position_ids, new_gate)


# speedup vs baseline: 1.0238x; 1.0238x over previous
import functools
import math

import jax
import jax.numpy as jnp
from jax import lax
from jax.experimental import pallas as pl
from jax.experimental.pallas import tpu as pltpu
from jax.experimental.pallas import tpu_sc as plsc

B, S, D = 2, 2048, 1024
H, KV, DH = 16, 8, 64
E, TOPK, F = 8, 2, 2048
THETA, EPS = 10000.0, 1e-6
T = B * S
A = T * TOPK
BLK = 256
NB = (A // BLK) + E - 1
PAD = NB * BLK
RB = 512
NRB = T // RB

HALF = DH // 2
_LOG_THETA = math.log(THETA)


def _qkv_body(x_ref, w_ref, ln_ref, cos_ref, sin_ref, out_ref):
    x = x_ref[...]
    ms = jnp.mean(x * x, axis=-1, keepdims=True)
    xn = x * jax.lax.rsqrt(ms + EPS) * ln_ref[...]
    qkv = jnp.dot(xn.astype(jnp.bfloat16), w_ref[...].astype(jnp.bfloat16),
                  preferred_element_type=jnp.float32)

    QK = (H + KV) * DH
    NH = QK // DH
    qk = qkv[:, :QK]
    cosv = jnp.concatenate([cos_ref[...]] * NH, axis=1)
    sinv = jnp.concatenate([sin_ref[...]] * NH, axis=1)
    col = jax.lax.broadcasted_iota(jnp.int32, (RB, QK), 1)
    rolled_p = jnp.concatenate([qk[:, HALF:], qk[:, :HALF]], axis=1)
    rolled_m = jnp.concatenate([qk[:, -HALF:], qk[:, :-HALF]], axis=1)
    first_half = (col % DH) < HALF
    rh = jnp.where(first_half, -rolled_p, rolled_m)
    qk_rot = qk * cosv + rh * sinv
    out_ref[...] = jnp.concatenate([qk_rot, qkv[:, QK:]], axis=1)


def _qkv_call(x2d, wqkv, ln1_w, cos_t, sin_t):
    nsb = S // RB
    return pl.pallas_call(
        _qkv_body,
        grid=(NRB,),
        in_specs=[
            pl.BlockSpec((RB, D), lambda i: (i, 0)),
            pl.BlockSpec((D, (H + 2 * KV) * DH), lambda i: (0, 0)),
            pl.BlockSpec((1, D), lambda i: (0, 0)),
            pl.BlockSpec((RB, DH), lambda i: (i % nsb, 0)),
            pl.BlockSpec((RB, DH), lambda i: (i % nsb, 0)),
        ],
        out_specs=pl.BlockSpec((RB, (H + 2 * KV) * DH), lambda i: (i, 0)),
        out_shape=jax.ShapeDtypeStruct((T, (H + 2 * KV) * DH), jnp.float32),
    )(x2d, wqkv, ln1_w, cos_t, sin_t)


QBLK = 512
NQB = S // QBLK
KBLK = 512


def _attn_body(q_ref, k_ref, v_ref, o_ref):
    iq = pl.program_id(2)
    q = q_ref[0].astype(jnp.bfloat16)
    lane = jax.lax.broadcasted_iota(jnp.int32, (KBLK, 2 * DH), 1)
    is_v = lane < DH
    is_one = lane == DH

    def vaug(j):
        v = v_ref[0, pl.ds(j * KBLK, KBLK), :]
        vb = jnp.concatenate([v, v], axis=1)
        return jnp.where(is_v, vb, jnp.where(is_one, 1.0, 0.0)
                         ).astype(jnp.bfloat16)

    def scores(j):
        k = k_ref[0, pl.ds(j * KBLK, KBLK), :].astype(jnp.bfloat16)
        s = jax.lax.dot_general(q, k, (((1,), (1,)), ((), ())),
                                preferred_element_type=jnp.float32)
        return s * (1.0 / 8.0)

    def update(s, j, carry):
        m, acc = carry
        mj = jnp.max(s, axis=-1, keepdims=True)
        mn = jnp.maximum(m, mj)
        p = jnp.exp((s - mn).astype(jnp.bfloat16))
        pv = jnp.dot(p, vaug(j), preferred_element_type=jnp.float32)
        return mn, acc * jnp.exp(m - mn) + pv

    def step(j, carry):
        return update(scores(j), j, carry)

    m0 = jnp.full((QBLK, 1), -1e30, jnp.float32)
    a0 = jnp.zeros((QBLK, 2 * DH), jnp.float32)
    nkb = iq * (QBLK // KBLK)
    m, acc = jax.lax.fori_loop(0, nkb, step, (m0, a0))

    sd = scores(nkb)
    qpos = iq * QBLK + jax.lax.broadcasted_iota(jnp.int32, (QBLK, KBLK), 0)
    kloc = nkb * KBLK + jax.lax.broadcasted_iota(jnp.int32, (QBLK, KBLK), 1)
    sd = jnp.where(qpos >= kloc, sd, jnp.float32(-1e9))
    m, acc = update(sd, nkb, (m, acc))

    o_ref[0] = acc[:, :DH] / acc[:, DH:DH + 1]


def _attn_call(q3, k3, v3):
    return pl.pallas_call(
        _attn_body,
        grid=(B, H, NQB),
        in_specs=[
            pl.BlockSpec((1, QBLK, DH), lambda b, h, iq: (h, b * NQB + iq, 0)),
            pl.BlockSpec((1, S, DH), lambda b, h, iq: (h // 2, b, 0)),
            pl.BlockSpec((1, S, DH), lambda b, h, iq: (h // 2, b, 0)),
        ],
        out_specs=pl.BlockSpec((1, QBLK, DH),
                               lambda b, h, iq: (h, b * NQB + iq, 0)),
        out_shape=jax.ShapeDtypeStruct((H, T, DH), jnp.float32),
    )(q3, k3, v3)


def _oproj_body(o_ref, wo_ref, res_ref, ln_ref, wg_ref,
                h1_ref, xn2_ref, logits_ref, route_ref):
    h1 = res_ref[...] + jnp.dot(o_ref[...].astype(jnp.bfloat16),
                                wo_ref[...].astype(jnp.bfloat16),
                                preferred_element_type=jnp.float32)
    h1_ref[...] = h1
    ms = jnp.mean(h1 * h1, axis=-1, keepdims=True)
    xn2 = h1 * jax.lax.rsqrt(ms + EPS) * ln_ref[...]
    xn2_ref[...] = xn2
    logits = jnp.dot(xn2.astype(jnp.bfloat16),
                     wg_ref[...].astype(jnp.bfloat16),
                     preferred_element_type=jnp.float32)
    logits_ref[...] = logits
    lm = jnp.max(logits, axis=-1, keepdims=True)
    ex = jnp.exp(logits - lm)
    probs = ex / jnp.sum(ex, axis=-1, keepdims=True)
    ioe = jax.lax.broadcasted_iota(jnp.int32, (RB, E), 1)
    m1 = jnp.max(probs, axis=-1, keepdims=True)
    e1 = jnp.min(jnp.where(probs == m1, ioe, E), axis=-1, keepdims=True)
    probs2 = jnp.where(ioe == e1, jnp.float32(-1.0), probs)
    m2 = jnp.max(probs2, axis=-1, keepdims=True)
    e2 = jnp.min(jnp.where(probs2 == m2, ioe, E), axis=-1, keepdims=True)
    denom = m1 + m2
    w1 = m1 / denom
    w2 = m2 / denom
    route_ref[...] = jnp.concatenate(
        [e1.astype(jnp.float32), e2.astype(jnp.float32), w1, w2,
         jnp.zeros((RB, 4), jnp.float32)], axis=1)


def _oproj_call(o, wo, res, ln2_w, wg):
    return pl.pallas_call(
        _oproj_body,
        grid=(NRB,),
        in_specs=[
            pl.BlockSpec((RB, H * DH), lambda i: (i, 0)),
            pl.BlockSpec((H * DH, D), lambda i: (0, 0)),
            pl.BlockSpec((RB, D), lambda i: (i, 0)),
            pl.BlockSpec((1, D), lambda i: (0, 0)),
            pl.BlockSpec((D, E), lambda i: (0, 0)),
        ],
        out_specs=[
            pl.BlockSpec((RB, D), lambda i: (i, 0)),
            pl.BlockSpec((RB, D), lambda i: (i, 0)),
            pl.BlockSpec((RB, E), lambda i: (i, 0)),
            pl.BlockSpec((RB, E), lambda i: (i, 0)),
        ],
        out_shape=[
            jax.ShapeDtypeStruct((T, D), jnp.float32),
            jax.ShapeDtypeStruct((T, D), jnp.float32),
            jax.ShapeDtypeStruct((T, E), jnp.float32),
            jax.ShapeDtypeStruct((T, E), jnp.float32),
        ],
    )(o, wo, res, ln2_w, wg)


def _gmm_body(be_ref, xs_ref, w1_ref, w3_ref, w2_ref, ws_ref, y_ref):
    del be_ref
    xs = xs_ref[...].astype(jnp.bfloat16)
    a = jnp.dot(xs, w1_ref[0].astype(jnp.bfloat16),
                preferred_element_type=jnp.float32)
    b = jnp.dot(xs, w3_ref[0].astype(jnp.bfloat16),
                preferred_element_type=jnp.float32)
    h = (a / (1.0 + jnp.exp(-a))) * b
    y = jnp.dot(h.astype(jnp.bfloat16), w2_ref[0].astype(jnp.bfloat16),
                preferred_element_type=jnp.float32)
    y_ref[...] = y * ws_ref[...]


def _gmm_call(xs, w1, w3, w2, ws, block_expert):
    grid_spec = pltpu.PrefetchScalarGridSpec(
        num_scalar_prefetch=1,
        grid=(NB,),
        in_specs=[
            pl.BlockSpec((BLK, D), lambda i, be: (i, 0)),
            pl.BlockSpec((1, D, F), lambda i, be: (be[i], 0, 0)),
            pl.BlockSpec((1, D, F), lambda i, be: (be[i], 0, 0)),
            pl.BlockSpec((1, F, D), lambda i, be: (be[i], 0, 0)),
            pl.BlockSpec((BLK, 1), lambda i, be: (i, 0)),
        ],
        out_specs=pl.BlockSpec((BLK, D), lambda i, be: (i, 0)),
    )
    return pl.pallas_call(
        _gmm_body,
        grid_spec=grid_spec,
        out_shape=jax.ShapeDtypeStruct((PAD, D), jnp.float32),
    )(block_expert, xs, w1, w3, w2, ws)


_SC_NC, _SC_NS = 2, 16
_SC_NW = _SC_NC * _SC_NS
GCH = (PAD // _SC_NW) // 3
TCH = (T // _SC_NW) // 4


def _sc_gather_rows(x_hbm, idx_hbm, out_hbm, idx_v, rows_v, sem):
    wid = lax.axis_index("s") * _SC_NC + lax.axis_index("c")
    base = wid * (PAD // _SC_NW)
    for c in range(3):
        off = base + c * GCH
        pltpu.sync_copy(idx_hbm.at[pl.ds(off, GCH)], idx_v)
        pltpu.async_copy(x_hbm.at[idx_v], rows_v, sem).wait()
        pltpu.sync_copy(rows_v, out_hbm.at[pl.ds(off, GCH)])


def _sc_gather_call(xn2, tok_src):
    mesh = plsc.VectorSubcoreMesh(core_axis_name="c", subcore_axis_name="s")
    f = functools.partial(
        pl.kernel, mesh=mesh,
        out_type=jax.ShapeDtypeStruct((PAD, D), jnp.float32),
        scratch_types=[
            pltpu.VMEM((GCH,), jnp.int32),
            pltpu.VMEM((GCH, D), jnp.float32),
            pltpu.SemaphoreType.DMA,
        ],
    )(_sc_gather_rows)
    return f(xn2, tok_src)


def _sc_combine(h1_hbm, y_hbm, p1_hbm, p2_hbm, out_hbm,
                i1_v, i2_v, r1_v, r2_v, h_v, sem):
    wid = lax.axis_index("s") * _SC_NC + lax.axis_index("c")
    base = wid * (T // _SC_NW)
    for c in range(4):
        off = base + c * TCH
        pltpu.sync_copy(p1_hbm.at[pl.ds(off, TCH)], i1_v)
        pltpu.sync_copy(p2_hbm.at[pl.ds(off, TCH)], i2_v)
        pltpu.async_copy(y_hbm.at[i1_v], r1_v, sem).wait()
        pltpu.async_copy(y_hbm.at[i2_v], r2_v, sem).wait()
        pltpu.sync_copy(h1_hbm.at[pl.ds(off, TCH)], h_v)

        def add_step(i, _):
            r = i // (D // 16)
            col = (i % (D // 16)) * 16
            h_v[r, pl.ds(col, 16)] = (h_v[r, pl.ds(col, 16)]
                                      + r1_v[r, pl.ds(col, 16)]
                                      + r2_v[r, pl.ds(col, 16)])
            return 0

        lax.fori_loop(0, TCH * (D // 16), add_step, 0, unroll=4)
        pltpu.sync_copy(h_v, out_hbm.at[pl.ds(off, TCH)])


def _sc_combine_call(h1, y, p1, p2):
    mesh = plsc.VectorSubcoreMesh(core_axis_name="c", subcore_axis_name="s")
    f = functools.partial(
        pl.kernel, mesh=mesh,
        out_type=jax.ShapeDtypeStruct((T, D), jnp.float32),
        scratch_types=[
            pltpu.VMEM((TCH,), jnp.int32),
            pltpu.VMEM((TCH,), jnp.int32),
            pltpu.VMEM((TCH, D), jnp.float32),
            pltpu.VMEM((TCH, D), jnp.float32),
            pltpu.VMEM((TCH, D), jnp.float32),
            pltpu.SemaphoreType.DMA,
        ],
    )(_sc_combine)
    return f(h1, y, p1, p2)


def _route_metadata(route):
    sel = jnp.stack([route[:, 0], route[:, 1]], axis=1).astype(jnp.int32)
    wts = jnp.stack([route[:, 2], route[:, 3]], axis=1)
    self = sel.reshape(A)
    wflat = wts.reshape(A)
    onehot = (self[:, None] == jnp.arange(E, dtype=jnp.int32)[None, :])
    onehot = onehot.astype(jnp.int32)
    ranks = jnp.cumsum(onehot, axis=0) - onehot
    rank_j = jnp.sum(ranks * onehot, axis=1)
    counts = jnp.sum(onehot, axis=0)
    pcounts = ((counts + BLK - 1) // BLK) * BLK
    poff = jnp.concatenate([jnp.zeros((1,), jnp.int32),
                            jnp.cumsum(pcounts)[:-1].astype(jnp.int32)])
    pos = poff[self] + rank_j
    tok = jnp.arange(A, dtype=jnp.int32) // TOPK
    tok_src = jnp.zeros((PAD,), jnp.int32).at[pos].set(tok)
    ws = jnp.zeros((PAD,), jnp.float32).at[pos].set(wflat)
    block_expert = jnp.repeat(jnp.arange(E, dtype=jnp.int32),
                              pcounts // BLK, total_repeat_length=NB)
    return pos, tok_src, ws, block_expert


def kernel(hidden_states, position_ids, gate_logits, ln1_w, ln2_w,
           Wq, Wk, Wv, Wo, Wg, w1, w3, w2):
    x2d = hidden_states.reshape(T, D)
    wqkv = jnp.concatenate([Wq, Wk, Wv], axis=1)

    inv = 1.0 / (THETA ** (jnp.arange(0, DH, 2, dtype=jnp.float32) / DH))
    ang = jnp.arange(S, dtype=jnp.float32)[:, None] * inv[None, :]
    cos_t = jnp.concatenate([jnp.cos(ang), jnp.cos(ang)], axis=1)
    sin_t = jnp.concatenate([jnp.sin(ang), jnp.sin(ang)], axis=1)

    qkv = _qkv_call(x2d, wqkv, ln1_w.reshape(1, D), cos_t, sin_t)
    q3 = qkv[:, :H * DH].reshape(T, H, DH).transpose(1, 0, 2)
    k3 = qkv[:, H * DH:(H + KV) * DH].reshape(T, KV, DH).transpose(1, 0, 2)
    v3 = qkv[:, (H + KV) * DH:].reshape(T, KV, DH).transpose(1, 0, 2)
    o3 = _attn_call(q3, k3, v3)
    o = o3.transpose(1, 0, 2).reshape(T, H * DH)
    h1, xn2, logits, route = _oproj_call(o, Wo, x2d, ln2_w.reshape(1, D), Wg)

    pos, tok_src, ws, block_expert = _route_metadata(route)
    xs = _sc_gather_call(xn2, tok_src)
    y = _gmm_call(xs, w1, w3, w2, ws.reshape(PAD, 1), block_expert)

    p = pos.reshape(T, TOPK)
    out2d = _sc_combine_call(h1, y, p[:, 0], p[:, 1])

    out = out2d.reshape(B, S, D)
    new_gate = gate_logits.at[0].set(logits)
    return (out, position_ids, new_gate)
